# Initial kernel scaffold; baseline (speedup 1.0000x reference)
import jax
import jax.numpy as jnp
from jax.experimental import pallas as pl


def kernel(input, adj_t, W0, b0, W1, b1, W2, b2, g0, be0, g1, be1):
    raise NotImplementedError("write your pallas kernel here")


# speedup vs baseline: 9.7737x; 9.7737x over previous
import functools

import jax
import jax.numpy as jnp
from jax import lax
from jax.experimental import pallas as pl
from jax.experimental.pallas import tpu as pltpu
from jax.experimental.pallas import tpu_sc as plsc

F32 = jnp.float32
EPS = 1e-5
NC = 2
NS = 16


def _sc_mesh():
    return plsc.VectorSubcoreMesh(
        core_axis_name="c", subcore_axis_name="s", num_cores=NC, num_subcores=NS
    )


@functools.lru_cache(maxsize=None)
def _make_deg(E, NPAD, CH):
    NW = NC * NS
    EPT = E // NW
    NCH = EPT // CH
    RPC = NPAD // NS

    @functools.partial(
        pl.kernel,
        out_type=jax.ShapeDtypeStruct((NC, NPAD, 128), F32),
        mesh=_sc_mesh(),
        scratch_types=[
            pltpu.VMEM((CH,), jnp.int32),
            pltpu.VMEM((CH, 128), F32),
            pltpu.VMEM_SHARED((NPAD, 128), F32),
        ],
    )
    def deg_kernel(dst_hbm, ones_hbm, zcol_hbm, outp, didx, ones_v, acc):
        cid = lax.axis_index("c")
        sid = lax.axis_index("s")
        wid = cid * NS + sid
        r0 = sid * RPC
        pltpu.sync_copy(zcol_hbm.at[pl.ds(r0, RPC)], acc.at[pl.ds(r0, RPC)])
        pltpu.sync_copy(ones_hbm, ones_v)
        plsc.subcore_barrier()
        ebase = wid * EPT

        def body(c, carry):
            b = pl.multiple_of(ebase + c * CH, 8)
            pltpu.sync_copy(dst_hbm.at[pl.ds(b, CH)], didx)
            pltpu.sync_copy(ones_v, acc.at[didx], add=True)
            return carry

        lax.fori_loop(0, NCH, body, 0)
        plsc.subcore_barrier()
        pltpu.sync_copy(acc.at[pl.ds(r0, RPC)], outp.at[cid, pl.ds(r0, RPC)])

    return deg_kernel


@functools.lru_cache(maxsize=None)
def _make_agg(E, NPAD, D, CH):
    NW = NC * NS
    EPT = E // NW
    NCH = EPT // CH
    RPC = NPAD // NS

    @functools.partial(
        pl.kernel,
        out_type=jax.ShapeDtypeStruct((NC, NPAD, D), F32),
        mesh=_sc_mesh(),
        scratch_types=[
            pltpu.VMEM((CH,), jnp.int32),
            pltpu.VMEM((CH,), jnp.int32),
            pltpu.VMEM((CH, D), F32),
            pltpu.VMEM_SHARED((NPAD, D), F32),
            pltpu.SemaphoreType.DMA,
        ],
    )
    def agg_kernel(hp_hbm, src_hbm, dst_hbm, outp, sidx, didx, rows, acc, gsem):
        cid = lax.axis_index("c")
        sid = lax.axis_index("s")
        wid = cid * NS + sid
        r0 = sid * RPC

        pltpu.sync_copy(hp_hbm.at[pl.ds(r0, RPC)], acc.at[pl.ds(r0, RPC)])
        plsc.subcore_barrier()
        ebase = wid * EPT

        def body(c, carry):
            b = pl.multiple_of(ebase + c * CH, 8)
            pltpu.sync_copy(src_hbm.at[pl.ds(b, CH)], sidx)
            pltpu.sync_copy(dst_hbm.at[pl.ds(b, CH)], didx)
            pltpu.async_copy(hp_hbm.at[sidx], rows, gsem).wait()
            pltpu.sync_copy(rows, acc.at[didx], add=True)
            return carry

        lax.fori_loop(0, NCH, body, 0)
        plsc.subcore_barrier()
        pltpu.sync_copy(acc.at[pl.ds(r0, RPC)], outp.at[cid, pl.ds(r0, RPC)])

    return agg_kernel


def _matmul0(x, W, d0, d1, B):
    NPAD, D = x.shape

    def body(x_ref, w_ref, d0_ref, d1_ref, o_ref):
        dinv = lax.rsqrt(d0_ref[...][:, 0:1] + d1_ref[...][:, 0:1] + 1.0)
        h = jnp.dot(x_ref[...], w_ref[...], preferred_element_type=F32)
        o_ref[...] = h * dinv

    return pl.pallas_call(
        body,
        grid=(NPAD // B,),
        in_specs=[
            pl.BlockSpec((B, D), lambda i: (i, 0)),
            pl.BlockSpec((D, D), lambda i: (0, 0)),
            pl.BlockSpec((B, 8), lambda i: (i, 0)),
            pl.BlockSpec((B, 8), lambda i: (i, 0)),
        ],
        out_specs=pl.BlockSpec((B, D), lambda i: (i, 0)),
        out_shape=jax.ShapeDtypeStruct((NPAD, D), F32),
    )(x, W, d0, d1)


def _matmul_bn(t, s, ss, g, be, W, d0, d1, N, B):
    NPAD, D = t.shape
    inv_n = 1.0 / N

    def body(t_ref, s_ref, ss_ref, g_ref, be_ref, w_ref, d0_ref, d1_ref, o_ref):
        mu = s_ref[...] * inv_n
        var = ss_ref[...] * inv_n - mu * mu
        rstd = lax.rsqrt(var + EPS)
        xb = g_ref[...] * (t_ref[...] - mu) * rstd + be_ref[...]
        xb = jnp.maximum(xb, 0.0)
        dinv = lax.rsqrt(d0_ref[...][:, 0:1] + d1_ref[...][:, 0:1] + 1.0)
        h = jnp.dot(xb, w_ref[...], preferred_element_type=F32)
        o_ref[...] = h * dinv

    return pl.pallas_call(
        body,
        grid=(NPAD // B,),
        in_specs=[
            pl.BlockSpec((B, D), lambda i: (i, 0)),
            pl.BlockSpec((1, D), lambda i: (0, 0)),
            pl.BlockSpec((1, D), lambda i: (0, 0)),
            pl.BlockSpec((1, D), lambda i: (0, 0)),
            pl.BlockSpec((1, D), lambda i: (0, 0)),
            pl.BlockSpec((D, D), lambda i: (0, 0)),
            pl.BlockSpec((B, 8), lambda i: (i, 0)),
            pl.BlockSpec((B, 8), lambda i: (i, 0)),
        ],
        out_specs=pl.BlockSpec((B, D), lambda i: (i, 0)),
        out_shape=jax.ShapeDtypeStruct((NPAD, D), F32),
    )(t, s, ss, g, be, W, d0, d1)


def _combine(p0, p1, hp, d0, d1, bias, N, B):
    NPAD, D = p0.shape

    def body(p0_ref, p1_ref, hp_ref, d0_ref, d1_ref, b_ref, t_ref, s_ref, ss_ref):
        i = pl.program_id(0)
        dinv = lax.rsqrt(d0_ref[...][:, 0:1] + d1_ref[...][:, 0:1] + 1.0)
        t = (p0_ref[...] + p1_ref[...] - hp_ref[...]) * dinv + b_ref[...]
        t_ref[...] = t
        rows = lax.broadcasted_iota(jnp.int32, (B, 1), 0) + i * B
        tm = jnp.where(rows < N, t, 0.0)

        @pl.when(i == 0)
        def _():
            s_ref[...] = jnp.zeros_like(s_ref)
            ss_ref[...] = jnp.zeros_like(ss_ref)

        s_ref[...] += jnp.sum(tm, axis=0, keepdims=True)
        ss_ref[...] += jnp.sum(tm * tm, axis=0, keepdims=True)

    return pl.pallas_call(
        body,
        grid=(NPAD // B,),
        in_specs=[
            pl.BlockSpec((B, D), lambda i: (i, 0)),
            pl.BlockSpec((B, D), lambda i: (i, 0)),
            pl.BlockSpec((B, D), lambda i: (i, 0)),
            pl.BlockSpec((B, 8), lambda i: (i, 0)),
            pl.BlockSpec((B, 8), lambda i: (i, 0)),
            pl.BlockSpec((1, D), lambda i: (0, 0)),
        ],
        out_specs=[
            pl.BlockSpec((B, D), lambda i: (i, 0)),
            pl.BlockSpec((1, D), lambda i: (0, 0)),
            pl.BlockSpec((1, D), lambda i: (0, 0)),
        ],
        out_shape=[
            jax.ShapeDtypeStruct((NPAD, D), F32),
            jax.ShapeDtypeStruct((1, D), F32),
            jax.ShapeDtypeStruct((1, D), F32),
        ],
    )(p0, p1, hp, d0, d1, bias)


def _final(p0, p1, hp, d0, d1, bias, B):
    NPAD, D = p0.shape

    def body(p0_ref, p1_ref, hp_ref, d0_ref, d1_ref, b_ref, y_ref):
        dinv = lax.rsqrt(d0_ref[...][:, 0:1] + d1_ref[...][:, 0:1] + 1.0)
        t = (p0_ref[...] + p1_ref[...] - hp_ref[...]) * dinv + b_ref[...]
        mx = jnp.max(t, axis=1, keepdims=True)
        lse = jnp.log(jnp.sum(jnp.exp(t - mx), axis=1, keepdims=True)) + mx
        y_ref[...] = t - lse

    return pl.pallas_call(
        body,
        grid=(NPAD // B,),
        in_specs=[
            pl.BlockSpec((B, D), lambda i: (i, 0)),
            pl.BlockSpec((B, D), lambda i: (i, 0)),
            pl.BlockSpec((B, D), lambda i: (i, 0)),
            pl.BlockSpec((B, 8), lambda i: (i, 0)),
            pl.BlockSpec((B, 8), lambda i: (i, 0)),
            pl.BlockSpec((1, D), lambda i: (0, 0)),
        ],
        out_specs=pl.BlockSpec((B, D), lambda i: (i, 0)),
        out_shape=jax.ShapeDtypeStruct((NPAD, D), F32),
    )(p0, p1, hp, d0, d1, bias)


def kernel(input, adj_t, W0, b0, W1, b1, W2, b2, g0, be0, g1, be1):
    N, D = input.shape
    E = adj_t.shape[1]
    NW = NC * NS
    NPAD = -(-N // (NW * 8)) * (NW * 8)
    EPT = E // NW
    assert E % NW == 0 and EPT % 8 == 0
    CH = max(c for c in range(8, 129, 8) if EPT % c == 0)
    B = 1024 if NPAD % 1024 == 0 else 512

    src = adj_t[0]
    dst = adj_t[1]
    xpad = jnp.pad(input, ((0, NPAD - N), (0, 0)))
    zcol = jnp.zeros((NPAD, 128), F32)
    ones = jnp.ones((CH, 128), F32)

    deg_fn = _make_deg(E, NPAD, CH)
    agg_fn = _make_agg(E, NPAD, D, CH)

    degp = deg_fn(dst, ones, zcol)
    d0, d1 = degp[0, :, :8], degp[1, :, :8]

    b0r, b1r, b2r = (b.reshape(1, D) for b in (b0, b1, b2))
    g0r, g1r = g0.reshape(1, D), g1.reshape(1, D)
    be0r, be1r = be0.reshape(1, D), be1.reshape(1, D)

    hp = _matmul0(xpad, W0, d0, d1, B)
    p = agg_fn(hp, src, dst)
    t, s, ss = _combine(p[0], p[1], hp, d0, d1, b0r, N, B)
    hp = _matmul_bn(t, s, ss, g0r, be0r, W1, d0, d1, N, B)
    p = agg_fn(hp, src, dst)
    t, s, ss = _combine(p[0], p[1], hp, d0, d1, b1r, N, B)
    hp = _matmul_bn(t, s, ss, g1r, be1r, W2, d0, d1, N, B)
    p = agg_fn(hp, src, dst)
    y = _final(p[0], p[1], hp, d0, d1, b2r, B)
    return y[:N]


# speedup vs baseline: 14.2992x; 1.4630x over previous
import functools

import jax
import jax.numpy as jnp
from jax import lax
from jax.experimental import pallas as pl
from jax.experimental.pallas import tpu as pltpu
from jax.experimental.pallas import tpu_sc as plsc

F32 = jnp.float32
EPS = 1e-5
NC = 2
NS = 16


def _sc_mesh():
    return plsc.VectorSubcoreMesh(
        core_axis_name="c", subcore_axis_name="s", num_cores=NC, num_subcores=NS
    )


@functools.lru_cache(maxsize=None)
def _make_deg(E, NPAD, CH):
    NW = NC * NS
    EPT = E // NW
    NCH = EPT // CH
    RPC = NPAD // NS

    @functools.partial(
        pl.kernel,
        out_type=jax.ShapeDtypeStruct((NC, NPAD, 128), F32),
        mesh=_sc_mesh(),
        scratch_types=[
            pltpu.VMEM((CH,), jnp.int32),
            pltpu.VMEM((CH, 128), F32),
            pltpu.VMEM_SHARED((NPAD, 128), F32),
        ],
    )
    def deg_kernel(dst_hbm, ones_hbm, zcol_hbm, outp, didx, ones_v, acc):
        cid = lax.axis_index("c")
        sid = lax.axis_index("s")
        wid = cid * NS + sid
        r0 = sid * RPC
        pltpu.sync_copy(zcol_hbm.at[pl.ds(r0, RPC)], acc.at[pl.ds(r0, RPC)])
        pltpu.sync_copy(ones_hbm, ones_v)
        plsc.subcore_barrier()
        ebase = wid * EPT

        def body(c, carry):
            b = pl.multiple_of(ebase + c * CH, 8)
            pltpu.sync_copy(dst_hbm.at[pl.ds(b, CH)], didx)
            pltpu.sync_copy(ones_v, acc.at[didx], add=True)
            return carry

        lax.fori_loop(0, NCH, body, 0)
        plsc.subcore_barrier()
        pltpu.sync_copy(acc.at[pl.ds(r0, RPC)], outp.at[cid, pl.ds(r0, RPC)])

    return deg_kernel


@functools.lru_cache(maxsize=None)
def _make_agg(E, NPAD, D, CH):
    NW = NC * NS
    EPT = E // NW
    NCH = EPT // CH
    assert NCH % 2 == 1
    RPC = NPAD // NS

    @functools.partial(
        pl.kernel,
        out_type=jax.ShapeDtypeStruct((NC, NPAD, D), F32),
        mesh=_sc_mesh(),
        scratch_types=[
            pltpu.VMEM((CH,), jnp.int32),
            pltpu.VMEM((CH,), jnp.int32),
            pltpu.VMEM((CH,), jnp.int32),
            pltpu.VMEM((CH,), jnp.int32),
            pltpu.VMEM((CH, D), F32),
            pltpu.VMEM((CH, D), F32),
            pltpu.VMEM_SHARED((NPAD, D), F32),
            pltpu.SemaphoreType.DMA,
            pltpu.SemaphoreType.DMA,
        ],
    )
    def agg_kernel(hp_hbm, src_hbm, dst_hbm, outp,
                   sidx0, didx0, sidx1, didx1, rows0, rows1, acc, gsem0, gsem1):
        cid = lax.axis_index("c")
        sid = lax.axis_index("s")
        wid = cid * NS + sid
        r0 = sid * RPC

        pltpu.sync_copy(hp_hbm.at[pl.ds(r0, RPC)], acc.at[pl.ds(r0, RPC)])
        plsc.subcore_barrier()
        ebase = wid * EPT

        def fire(c, sidx, didx, rows, gsem):
            b = pl.multiple_of(ebase + c * CH, 8)
            pltpu.sync_copy(src_hbm.at[pl.ds(b, CH)], sidx)
            pltpu.sync_copy(dst_hbm.at[pl.ds(b, CH)], didx)
            pltpu.async_copy(hp_hbm.at[sidx], rows, gsem)

        def drain(sidx, didx, rows, gsem):
            pltpu.make_async_copy(hp_hbm.at[sidx], rows, gsem).wait()
            pltpu.sync_copy(rows, acc.at[didx], add=True)

        fire(0, sidx0, didx0, rows0, gsem0)

        def body(s, carry):
            c = 2 * s
            fire(c + 1, sidx1, didx1, rows1, gsem1)
            drain(sidx0, didx0, rows0, gsem0)
            fire(c + 2, sidx0, didx0, rows0, gsem0)
            drain(sidx1, didx1, rows1, gsem1)
            return carry

        lax.fori_loop(0, (NCH - 1) // 2, body, 0)
        drain(sidx0, didx0, rows0, gsem0)
        plsc.subcore_barrier()
        pltpu.sync_copy(acc.at[pl.ds(r0, RPC)], outp.at[cid, pl.ds(r0, RPC)])

    return agg_kernel


def _matmul0(x, W, d0, d1, B):
    NPAD, D = x.shape

    def body(x_ref, w_ref, d0_ref, d1_ref, o_ref):
        dinv = lax.rsqrt(d0_ref[...][:, 0:1] + d1_ref[...][:, 0:1] + 1.0)
        h = jnp.dot(x_ref[...], w_ref[...], preferred_element_type=F32)
        o_ref[...] = h * dinv

    return pl.pallas_call(
        body,
        grid=(NPAD // B,),
        in_specs=[
            pl.BlockSpec((B, D), lambda i: (i, 0)),
            pl.BlockSpec((D, D), lambda i: (0, 0)),
            pl.BlockSpec((B, 8), lambda i: (i, 0)),
            pl.BlockSpec((B, 8), lambda i: (i, 0)),
        ],
        out_specs=pl.BlockSpec((B, D), lambda i: (i, 0)),
        out_shape=jax.ShapeDtypeStruct((NPAD, D), F32),
    )(x, W, d0, d1)


def _matmul_bn(t, s, ss, g, be, W, d0, d1, N, B):
    NPAD, D = t.shape
    inv_n = 1.0 / N

    def body(t_ref, s_ref, ss_ref, g_ref, be_ref, w_ref, d0_ref, d1_ref, o_ref):
        mu = s_ref[...] * inv_n
        var = ss_ref[...] * inv_n - mu * mu
        rstd = lax.rsqrt(var + EPS)
        xb = g_ref[...] * (t_ref[...] - mu) * rstd + be_ref[...]
        xb = jnp.maximum(xb, 0.0)
        dinv = lax.rsqrt(d0_ref[...][:, 0:1] + d1_ref[...][:, 0:1] + 1.0)
        h = jnp.dot(xb, w_ref[...], preferred_element_type=F32)
        o_ref[...] = h * dinv

    return pl.pallas_call(
        body,
        grid=(NPAD // B,),
        in_specs=[
            pl.BlockSpec((B, D), lambda i: (i, 0)),
            pl.BlockSpec((1, D), lambda i: (0, 0)),
            pl.BlockSpec((1, D), lambda i: (0, 0)),
            pl.BlockSpec((1, D), lambda i: (0, 0)),
            pl.BlockSpec((1, D), lambda i: (0, 0)),
            pl.BlockSpec((D, D), lambda i: (0, 0)),
            pl.BlockSpec((B, 8), lambda i: (i, 0)),
            pl.BlockSpec((B, 8), lambda i: (i, 0)),
        ],
        out_specs=pl.BlockSpec((B, D), lambda i: (i, 0)),
        out_shape=jax.ShapeDtypeStruct((NPAD, D), F32),
    )(t, s, ss, g, be, W, d0, d1)


def _combine(p0, p1, hp, d0, d1, bias, N, B):
    NPAD, D = p0.shape

    def body(p0_ref, p1_ref, hp_ref, d0_ref, d1_ref, b_ref, t_ref, s_ref, ss_ref):
        i = pl.program_id(0)
        dinv = lax.rsqrt(d0_ref[...][:, 0:1] + d1_ref[...][:, 0:1] + 1.0)
        t = (p0_ref[...] + p1_ref[...] - hp_ref[...]) * dinv + b_ref[...]
        t_ref[...] = t
        rows = lax.broadcasted_iota(jnp.int32, (B, 1), 0) + i * B
        tm = jnp.where(rows < N, t, 0.0)

        @pl.when(i == 0)
        def _():
            s_ref[...] = jnp.zeros_like(s_ref)
            ss_ref[...] = jnp.zeros_like(ss_ref)

        s_ref[...] += jnp.sum(tm, axis=0, keepdims=True)
        ss_ref[...] += jnp.sum(tm * tm, axis=0, keepdims=True)

    return pl.pallas_call(
        body,
        grid=(NPAD // B,),
        in_specs=[
            pl.BlockSpec((B, D), lambda i: (i, 0)),
            pl.BlockSpec((B, D), lambda i: (i, 0)),
            pl.BlockSpec((B, D), lambda i: (i, 0)),
            pl.BlockSpec((B, 8), lambda i: (i, 0)),
            pl.BlockSpec((B, 8), lambda i: (i, 0)),
            pl.BlockSpec((1, D), lambda i: (0, 0)),
        ],
        out_specs=[
            pl.BlockSpec((B, D), lambda i: (i, 0)),
            pl.BlockSpec((1, D), lambda i: (0, 0)),
            pl.BlockSpec((1, D), lambda i: (0, 0)),
        ],
        out_shape=[
            jax.ShapeDtypeStruct((NPAD, D), F32),
            jax.ShapeDtypeStruct((1, D), F32),
            jax.ShapeDtypeStruct((1, D), F32),
        ],
    )(p0, p1, hp, d0, d1, bias)


def _final(p0, p1, hp, d0, d1, bias, B):
    NPAD, D = p0.shape

    def body(p0_ref, p1_ref, hp_ref, d0_ref, d1_ref, b_ref, y_ref):
        dinv = lax.rsqrt(d0_ref[...][:, 0:1] + d1_ref[...][:, 0:1] + 1.0)
        t = (p0_ref[...] + p1_ref[...] - hp_ref[...]) * dinv + b_ref[...]
        mx = jnp.max(t, axis=1, keepdims=True)
        lse = jnp.log(jnp.sum(jnp.exp(t - mx), axis=1, keepdims=True)) + mx
        y_ref[...] = t - lse

    return pl.pallas_call(
        body,
        grid=(NPAD // B,),
        in_specs=[
            pl.BlockSpec((B, D), lambda i: (i, 0)),
            pl.BlockSpec((B, D), lambda i: (i, 0)),
            pl.BlockSpec((B, D), lambda i: (i, 0)),
            pl.BlockSpec((B, 8), lambda i: (i, 0)),
            pl.BlockSpec((B, 8), lambda i: (i, 0)),
            pl.BlockSpec((1, D), lambda i: (0, 0)),
        ],
        out_specs=pl.BlockSpec((B, D), lambda i: (i, 0)),
        out_shape=jax.ShapeDtypeStruct((NPAD, D), F32),
    )(p0, p1, hp, d0, d1, bias)


def kernel(input, adj_t, W0, b0, W1, b1, W2, b2, g0, be0, g1, be1):
    N, D = input.shape
    E = adj_t.shape[1]
    NW = NC * NS
    NPAD = -(-N // (NW * 8)) * (NW * 8)
    EPT = E // NW
    assert E % NW == 0 and EPT % 8 == 0
    CH = max(c for c in range(8, 129, 8) if EPT % c == 0)
    B = 1024 if NPAD % 1024 == 0 else 512

    src = adj_t[0]
    dst = adj_t[1]
    xpad = jnp.pad(input, ((0, NPAD - N), (0, 0)))
    zcol = jnp.zeros((NPAD, 128), F32)
    ones = jnp.ones((CH, 128), F32)

    deg_fn = _make_deg(E, NPAD, CH)
    agg_fn = _make_agg(E, NPAD, D, CH)

    degp = deg_fn(dst, ones, zcol)
    d0, d1 = degp[0, :, :8], degp[1, :, :8]

    b0r, b1r, b2r = (b.reshape(1, D) for b in (b0, b1, b2))
    g0r, g1r = g0.reshape(1, D), g1.reshape(1, D)
    be0r, be1r = be0.reshape(1, D), be1.reshape(1, D)

    hp = _matmul0(xpad, W0, d0, d1, B)
    p = agg_fn(hp, src, dst)
    t, s, ss = _combine(p[0], p[1], hp, d0, d1, b0r, N, B)
    hp = _matmul_bn(t, s, ss, g0r, be0r, W1, d0, d1, N, B)
    p = agg_fn(hp, src, dst)
    t, s, ss = _combine(p[0], p[1], hp, d0, d1, b1r, N, B)
    hp = _matmul_bn(t, s, ss, g1r, be1r, W2, d0, d1, N, B)
    p = agg_fn(hp, src, dst)
    y = _final(p[0], p[1], hp, d0, d1, b2r, B)
    return y[:N]


# speedup vs baseline: 17.6479x; 1.2342x over previous
import functools

import jax
import jax.numpy as jnp
from jax import lax
from jax.experimental import pallas as pl
from jax.experimental.pallas import tpu as pltpu
from jax.experimental.pallas import tpu_sc as plsc

F32 = jnp.float32
EPS = 1e-5
NC = 2
NS = 16


def _sc_mesh():
    return plsc.VectorSubcoreMesh(
        core_axis_name="c", subcore_axis_name="s", num_cores=NC, num_subcores=NS
    )


@functools.lru_cache(maxsize=None)
def _make_deg(E, NPAD, CH):
    NW = NC * NS
    EPT = E // NW
    NCH = EPT // CH
    RPC = NPAD // NS

    @functools.partial(
        pl.kernel,
        out_type=jax.ShapeDtypeStruct((NC, NPAD, 128), F32),
        mesh=_sc_mesh(),
        scratch_types=[
            pltpu.VMEM((NCH, CH), jnp.int32),
            pltpu.VMEM((CH, 128), F32),
            pltpu.VMEM_SHARED((NPAD, 128), F32),
        ],
    )
    def deg_kernel(dst_hbm, ones_hbm, zcol_hbm, outp, didx_all, ones_v, acc):
        cid = lax.axis_index("c")
        sid = lax.axis_index("s")
        wid = cid * NS + sid
        r0 = sid * RPC
        pltpu.sync_copy(zcol_hbm.at[pl.ds(r0, RPC)], acc.at[pl.ds(r0, RPC)])
        pltpu.sync_copy(ones_hbm, ones_v)
        pltpu.sync_copy(dst_hbm.at[wid], didx_all)
        plsc.subcore_barrier()

        def body(c, carry):
            pltpu.sync_copy(ones_v, acc.at[didx_all.at[c]], add=True)
            return carry

        lax.fori_loop(0, NCH, body, 0)
        plsc.subcore_barrier()
        pltpu.sync_copy(acc.at[pl.ds(r0, RPC)], outp.at[cid, pl.ds(r0, RPC)])

    return deg_kernel


@functools.lru_cache(maxsize=None)
def _make_agg(E, NPAD, D, CH):
    NW = NC * NS
    EPT = E // NW
    NCH = EPT // CH
    RPC = NPAD // NS

    @functools.partial(
        pl.kernel,
        out_type=jax.ShapeDtypeStruct((NC, NPAD, D), F32),
        mesh=_sc_mesh(),
        scratch_types=[
            pltpu.VMEM((2, CH), jnp.int32),
            pltpu.VMEM((2, CH), jnp.int32),
            pltpu.VMEM((CH, D), F32),
            pltpu.VMEM((CH, D), F32),
            pltpu.SemaphoreType.DMA,
            pltpu.SemaphoreType.DMA,
            pltpu.SemaphoreType.DMA,
            pltpu.SemaphoreType.DMA,
            pltpu.VMEM_SHARED((NPAD, D), F32),
        ],
    )
    def agg_kernel(hp_hbm, adj_hbm, outp, i0, i1, r0b, r1b,
                   is0, is1, gs0, gs1, acc):
        ibuf = (i0, i1)
        isems = (is0, is1)
        rows = (r0b, r1b)
        gsems = (gs0, gs1)
        cid = lax.axis_index("c")
        sid = lax.axis_index("s")
        wid = cid * NS + sid
        r0 = sid * RPC

        pltpu.sync_copy(hp_hbm.at[pl.ds(r0, RPC)], acc.at[pl.ds(r0, RPC)])
        plsc.subcore_barrier()

        def fire_idx(c, j):
            pltpu.async_copy(adj_hbm.at[wid, c], ibuf[j], isems[j])

        def wait_idx(j):
            pltpu.make_async_copy(adj_hbm.at[wid, 0], ibuf[j], isems[j]).wait()

        def fire_gather(k):
            pltpu.async_copy(hp_hbm.at[ibuf[k].at[0]], rows[k], gsems[k])

        def wait_gather(k):
            pltpu.make_async_copy(hp_hbm.at[ibuf[k].at[0]], rows[k], gsems[k]).wait()

        def scatter(k):
            pltpu.sync_copy(rows[k], acc.at[ibuf[k].at[1]], add=True)

        def step(c, k):
            if c + 1 < NCH:
                wait_idx((k + 1) % 2)
                fire_gather((k + 1) % 2)
            wait_gather(k)
            scatter(k)
            if c + 2 < NCH:
                fire_idx(c + 2, k)

        def step_dyn(c, k):
            wait_idx((k + 1) % 2)
            fire_gather((k + 1) % 2)
            wait_gather(k)
            scatter(k)
            fire_idx(c + 2, k)

        fire_idx(0, 0)
        fire_idx(1, 1)
        wait_idx(0)
        fire_gather(0)

        def body(s, carry):
            c = 2 * s
            step_dyn(c, 0)
            step_dyn(c + 1, 1)
            return carry

        NTAIL = 3 if NCH % 2 == 1 else 4
        lax.fori_loop(0, (NCH - NTAIL) // 2, body, 0)
        for c in range(NCH - NTAIL, NCH):
            step(c, c % 2)
        plsc.subcore_barrier()
        pltpu.sync_copy(acc.at[pl.ds(r0, RPC)], outp.at[cid, pl.ds(r0, RPC)])

    return agg_kernel


def _matmul0(x, W, d0, d1, B):
    NPAD, D = x.shape

    def body(x_ref, w_ref, d0_ref, d1_ref, o_ref):
        dinv = lax.rsqrt(d0_ref[...][:, 0:1] + d1_ref[...][:, 0:1] + 1.0)
        h = jnp.dot(x_ref[...], w_ref[...], preferred_element_type=F32)
        o_ref[...] = h * dinv

    return pl.pallas_call(
        body,
        grid=(NPAD // B,),
        in_specs=[
            pl.BlockSpec((B, D), lambda i: (i, 0)),
            pl.BlockSpec((D, D), lambda i: (0, 0)),
            pl.BlockSpec((B, 8), lambda i: (i, 0)),
            pl.BlockSpec((B, 8), lambda i: (i, 0)),
        ],
        out_specs=pl.BlockSpec((B, D), lambda i: (i, 0)),
        out_shape=jax.ShapeDtypeStruct((NPAD, D), F32),
    )(x, W, d0, d1)


def _matmul_bn(t, s, ss, g, be, W, d0, d1, N, B):
    NPAD, D = t.shape
    inv_n = 1.0 / N

    def body(t_ref, s_ref, ss_ref, g_ref, be_ref, w_ref, d0_ref, d1_ref, o_ref):
        mu = s_ref[...] * inv_n
        var = ss_ref[...] * inv_n - mu * mu
        rstd = lax.rsqrt(var + EPS)
        xb = g_ref[...] * (t_ref[...] - mu) * rstd + be_ref[...]
        xb = jnp.maximum(xb, 0.0)
        dinv = lax.rsqrt(d0_ref[...][:, 0:1] + d1_ref[...][:, 0:1] + 1.0)
        h = jnp.dot(xb, w_ref[...], preferred_element_type=F32)
        o_ref[...] = h * dinv

    return pl.pallas_call(
        body,
        grid=(NPAD // B,),
        in_specs=[
            pl.BlockSpec((B, D), lambda i: (i, 0)),
            pl.BlockSpec((1, D), lambda i: (0, 0)),
            pl.BlockSpec((1, D), lambda i: (0, 0)),
            pl.BlockSpec((1, D), lambda i: (0, 0)),
            pl.BlockSpec((1, D), lambda i: (0, 0)),
            pl.BlockSpec((D, D), lambda i: (0, 0)),
            pl.BlockSpec((B, 8), lambda i: (i, 0)),
            pl.BlockSpec((B, 8), lambda i: (i, 0)),
        ],
        out_specs=pl.BlockSpec((B, D), lambda i: (i, 0)),
        out_shape=jax.ShapeDtypeStruct((NPAD, D), F32),
    )(t, s, ss, g, be, W, d0, d1)


def _combine(p0, p1, hp, d0, d1, bias, N, B):
    NPAD, D = p0.shape

    def body(p0_ref, p1_ref, hp_ref, d0_ref, d1_ref, b_ref, t_ref, s_ref, ss_ref):
        i = pl.program_id(0)
        dinv = lax.rsqrt(d0_ref[...][:, 0:1] + d1_ref[...][:, 0:1] + 1.0)
        t = (p0_ref[...] + p1_ref[...] - hp_ref[...]) * dinv + b_ref[...]
        t_ref[...] = t
        rows = lax.broadcasted_iota(jnp.int32, (B, 1), 0) + i * B
        tm = jnp.where(rows < N, t, 0.0)

        @pl.when(i == 0)
        def _():
            s_ref[...] = jnp.zeros_like(s_ref)
            ss_ref[...] = jnp.zeros_like(ss_ref)

        s_ref[...] += jnp.sum(tm, axis=0, keepdims=True)
        ss_ref[...] += jnp.sum(tm * tm, axis=0, keepdims=True)

    return pl.pallas_call(
        body,
        grid=(NPAD // B,),
        in_specs=[
            pl.BlockSpec((B, D), lambda i: (i, 0)),
            pl.BlockSpec((B, D), lambda i: (i, 0)),
            pl.BlockSpec((B, D), lambda i: (i, 0)),
            pl.BlockSpec((B, 8), lambda i: (i, 0)),
            pl.BlockSpec((B, 8), lambda i: (i, 0)),
            pl.BlockSpec((1, D), lambda i: (0, 0)),
        ],
        out_specs=[
            pl.BlockSpec((B, D), lambda i: (i, 0)),
            pl.BlockSpec((1, D), lambda i: (0, 0)),
            pl.BlockSpec((1, D), lambda i: (0, 0)),
        ],
        out_shape=[
            jax.ShapeDtypeStruct((NPAD, D), F32),
            jax.ShapeDtypeStruct((1, D), F32),
            jax.ShapeDtypeStruct((1, D), F32),
        ],
    )(p0, p1, hp, d0, d1, bias)


def _final(p0, p1, hp, d0, d1, bias, B):
    NPAD, D = p0.shape

    def body(p0_ref, p1_ref, hp_ref, d0_ref, d1_ref, b_ref, y_ref):
        dinv = lax.rsqrt(d0_ref[...][:, 0:1] + d1_ref[...][:, 0:1] + 1.0)
        t = (p0_ref[...] + p1_ref[...] - hp_ref[...]) * dinv + b_ref[...]
        mx = jnp.max(t, axis=1, keepdims=True)
        lse = jnp.log(jnp.sum(jnp.exp(t - mx), axis=1, keepdims=True)) + mx
        y_ref[...] = t - lse

    return pl.pallas_call(
        body,
        grid=(NPAD // B,),
        in_specs=[
            pl.BlockSpec((B, D), lambda i: (i, 0)),
            pl.BlockSpec((B, D), lambda i: (i, 0)),
            pl.BlockSpec((B, D), lambda i: (i, 0)),
            pl.BlockSpec((B, 8), lambda i: (i, 0)),
            pl.BlockSpec((B, 8), lambda i: (i, 0)),
            pl.BlockSpec((1, D), lambda i: (0, 0)),
        ],
        out_specs=pl.BlockSpec((B, D), lambda i: (i, 0)),
        out_shape=jax.ShapeDtypeStruct((NPAD, D), F32),
    )(p0, p1, hp, d0, d1, bias)


def kernel(input, adj_t, W0, b0, W1, b1, W2, b2, g0, be0, g1, be1):
    N, D = input.shape
    E = adj_t.shape[1]
    NW = NC * NS
    NPAD = -(-N // (NW * 8)) * (NW * 8)
    EPT = E // NW
    assert E % NW == 0 and EPT % 8 == 0
    CH = max(c for c in range(8, 129, 8) if EPT % c == 0)
    CHD = CH
    B = 1024 if NPAD % 1024 == 0 else 512

    NCH = EPT // CH
    NCHD = EPT // CHD
    adjI = adj_t.reshape(2, NW, NCH, CH).transpose(1, 2, 0, 3)
    dst3 = adj_t[1].reshape(NW, NCHD, CHD)
    xpad = jnp.pad(input, ((0, NPAD - N), (0, 0)))
    zcol = jnp.zeros((NPAD, 128), F32)
    ones = jnp.ones((CHD, 128), F32)

    deg_fn = _make_deg(E, NPAD, CHD)
    agg_fn = _make_agg(E, NPAD, D, CH)

    degp = deg_fn(dst3, ones, zcol)
    d0, d1 = degp[0, :, :8], degp[1, :, :8]

    b0r, b1r, b2r = (b.reshape(1, D) for b in (b0, b1, b2))
    g0r, g1r = g0.reshape(1, D), g1.reshape(1, D)
    be0r, be1r = be0.reshape(1, D), be1.reshape(1, D)

    hp = _matmul0(xpad, W0, d0, d1, B)
    p = agg_fn(hp, adjI)
    t, s, ss = _combine(p[0], p[1], hp, d0, d1, b0r, N, B)
    hp = _matmul_bn(t, s, ss, g0r, be0r, W1, d0, d1, N, B)
    p = agg_fn(hp, adjI)
    t, s, ss = _combine(p[0], p[1], hp, d0, d1, b1r, N, B)
    hp = _matmul_bn(t, s, ss, g1r, be1r, W2, d0, d1, N, B)
    p = agg_fn(hp, adjI)
    y = _final(p[0], p[1], hp, d0, d1, b2r, B)
    return y[:N]


# speedup vs baseline: 18.8559x; 1.0685x over previous
import functools

import jax
import jax.numpy as jnp
from jax import lax
from jax.experimental import pallas as pl
from jax.experimental.pallas import tpu as pltpu
from jax.experimental.pallas import tpu_sc as plsc

F32 = jnp.float32
EPS = 1e-5
NC = 2
NS = 16


def _sc_mesh():
    return plsc.VectorSubcoreMesh(
        core_axis_name="c", subcore_axis_name="s", num_cores=NC, num_subcores=NS
    )


@functools.lru_cache(maxsize=None)
def _make_deg(E, NPAD, CH):
    NW = NC * NS
    EPT = E // NW
    NCH = EPT // CH
    RPC = NPAD // NS

    @functools.partial(
        pl.kernel,
        out_type=jax.ShapeDtypeStruct((NC, NPAD, 128), F32),
        mesh=_sc_mesh(),
        scratch_types=[
            pltpu.VMEM((NCH, CH), jnp.int32),
            pltpu.VMEM((CH, 128), F32),
            pltpu.VMEM_SHARED((NPAD, 128), F32),
        ],
    )
    def deg_kernel(dst_hbm, ones_hbm, zcol_hbm, outp, didx_all, ones_v, acc):
        cid = lax.axis_index("c")
        sid = lax.axis_index("s")
        wid = cid * NS + sid
        r0 = sid * RPC
        pltpu.sync_copy(zcol_hbm.at[pl.ds(r0, RPC)], acc.at[pl.ds(r0, RPC)])
        pltpu.sync_copy(ones_hbm, ones_v)
        pltpu.sync_copy(dst_hbm.at[wid], didx_all)
        plsc.subcore_barrier()

        def body(c, carry):
            pltpu.sync_copy(ones_v, acc.at[didx_all.at[c]], add=True)
            return carry

        lax.fori_loop(0, NCH, body, 0)
        plsc.subcore_barrier()
        pltpu.sync_copy(acc.at[pl.ds(r0, RPC)], outp.at[cid, pl.ds(r0, RPC)])

    return deg_kernel


@functools.lru_cache(maxsize=None)
def _make_agg(E, NPAD, D, CH):
    NW = NC * NS
    EPT = E // NW
    NCH = EPT // CH
    RPC = NPAD // NS

    @functools.partial(
        pl.kernel,
        out_type=jax.ShapeDtypeStruct((NC, NPAD, D), F32),
        mesh=_sc_mesh(),
        scratch_types=[
            pltpu.VMEM((2, CH), jnp.int32),
            pltpu.VMEM((2, CH), jnp.int32),
            pltpu.VMEM((CH, D), F32),
            pltpu.VMEM((CH, D), F32),
            pltpu.SemaphoreType.DMA,
            pltpu.SemaphoreType.DMA,
            pltpu.SemaphoreType.DMA,
            pltpu.SemaphoreType.DMA,
            pltpu.VMEM_SHARED((NPAD, D), F32),
        ],
    )
    def agg_kernel(hp_hbm, adj_hbm, outp, i0, i1, r0b, r1b,
                   is0, is1, gs0, gs1, acc):
        ibuf = (i0, i1)
        isems = (is0, is1)
        rows = (r0b, r1b)
        gsems = (gs0, gs1)
        cid = lax.axis_index("c")
        sid = lax.axis_index("s")
        wid = cid * NS + sid
        r0 = sid * RPC

        pltpu.sync_copy(hp_hbm.at[pl.ds(r0, RPC)], acc.at[pl.ds(r0, RPC)])
        plsc.subcore_barrier()

        def fire_idx(c, j):
            pltpu.async_copy(adj_hbm.at[wid, c], ibuf[j], isems[j])

        def wait_idx(j):
            pltpu.make_async_copy(adj_hbm.at[wid, 0], ibuf[j], isems[j]).wait()

        def fire_gather(k):
            pltpu.async_copy(hp_hbm.at[ibuf[k].at[0]], rows[k], gsems[k])

        def wait_gather(k):
            pltpu.make_async_copy(hp_hbm.at[ibuf[k].at[0]], rows[k], gsems[k]).wait()

        def scatter(k):
            pltpu.sync_copy(rows[k], acc.at[ibuf[k].at[1]], add=True)

        def step(c, k):
            if c + 1 < NCH:
                wait_idx((k + 1) % 2)
                fire_gather((k + 1) % 2)
            wait_gather(k)
            scatter(k)
            if c + 2 < NCH:
                fire_idx(c + 2, k)

        def step_dyn(c, k):
            wait_idx((k + 1) % 2)
            fire_gather((k + 1) % 2)
            wait_gather(k)
            scatter(k)
            fire_idx(c + 2, k)

        fire_idx(0, 0)
        fire_idx(1, 1)
        wait_idx(0)
        fire_gather(0)

        def body(s, carry):
            c = 2 * s
            step_dyn(c, 0)
            step_dyn(c + 1, 1)
            return carry

        NTAIL = 3 if NCH % 2 == 1 else 4
        lax.fori_loop(0, (NCH - NTAIL) // 2, body, 0)
        for c in range(NCH - NTAIL, NCH):
            step(c, c % 2)
        plsc.subcore_barrier()
        pltpu.sync_copy(acc.at[pl.ds(r0, RPC)], outp.at[cid, pl.ds(r0, RPC)])

    return agg_kernel


def _matmul0(x, W, dp, B):
    NPAD, D = x.shape

    def body(x_ref, w_ref, dp_ref, o_ref):
        dp = dp_ref[...]
        dinv = lax.rsqrt(dp[0][:, 0:1] + dp[1][:, 0:1] + 1.0)
        h = jnp.dot(x_ref[...], w_ref[...], preferred_element_type=F32)
        o_ref[...] = h * dinv

    return pl.pallas_call(
        body,
        grid=(NPAD // B,),
        in_specs=[
            pl.BlockSpec((B, D), lambda i: (i, 0)),
            pl.BlockSpec((D, D), lambda i: (0, 0)),
            pl.BlockSpec((2, B, 8), lambda i: (0, i, 0)),
        ],
        out_specs=pl.BlockSpec((B, D), lambda i: (i, 0)),
        out_shape=jax.ShapeDtypeStruct((NPAD, D), F32),
    )(x, W, dp)


def _matmul_bn(t, s, ss, g, be, W, dp, N, B):
    NPAD, D = t.shape
    inv_n = 1.0 / N

    def body(t_ref, s_ref, ss_ref, g_ref, be_ref, w_ref, dp_ref, o_ref):
        mu = s_ref[...] * inv_n
        var = ss_ref[...] * inv_n - mu * mu
        rstd = lax.rsqrt(var + EPS)
        xb = g_ref[...] * (t_ref[...] - mu) * rstd + be_ref[...]
        xb = jnp.maximum(xb, 0.0)
        dp = dp_ref[...]
        dinv = lax.rsqrt(dp[0][:, 0:1] + dp[1][:, 0:1] + 1.0)
        h = jnp.dot(xb, w_ref[...], preferred_element_type=F32)
        o_ref[...] = h * dinv

    return pl.pallas_call(
        body,
        grid=(NPAD // B,),
        in_specs=[
            pl.BlockSpec((B, D), lambda i: (i, 0)),
            pl.BlockSpec((1, D), lambda i: (0, 0)),
            pl.BlockSpec((1, D), lambda i: (0, 0)),
            pl.BlockSpec((1, D), lambda i: (0, 0)),
            pl.BlockSpec((1, D), lambda i: (0, 0)),
            pl.BlockSpec((D, D), lambda i: (0, 0)),
            pl.BlockSpec((2, B, 8), lambda i: (0, i, 0)),
        ],
        out_specs=pl.BlockSpec((B, D), lambda i: (i, 0)),
        out_shape=jax.ShapeDtypeStruct((NPAD, D), F32),
    )(t, s, ss, g, be, W, dp)


def _combine(p, hp, dp, bias, N, B):
    _, NPAD, D = p.shape

    def body(p_ref, hp_ref, dp_ref, b_ref, t_ref, s_ref, ss_ref):
        i = pl.program_id(0)
        pv = p_ref[...]
        dp = dp_ref[...]
        dinv = lax.rsqrt(dp[0][:, 0:1] + dp[1][:, 0:1] + 1.0)
        t = (pv[0] + pv[1] - hp_ref[...]) * dinv + b_ref[...]
        t_ref[...] = t
        rows = lax.broadcasted_iota(jnp.int32, (B, 1), 0) + i * B
        tm = jnp.where(rows < N, t, 0.0)

        @pl.when(i == 0)
        def _():
            s_ref[...] = jnp.zeros_like(s_ref)
            ss_ref[...] = jnp.zeros_like(ss_ref)

        s_ref[...] += jnp.sum(tm, axis=0, keepdims=True)
        ss_ref[...] += jnp.sum(tm * tm, axis=0, keepdims=True)

    return pl.pallas_call(
        body,
        grid=(NPAD // B,),
        in_specs=[
            pl.BlockSpec((2, B, D), lambda i: (0, i, 0)),
            pl.BlockSpec((B, D), lambda i: (i, 0)),
            pl.BlockSpec((2, B, 8), lambda i: (0, i, 0)),
            pl.BlockSpec((1, D), lambda i: (0, 0)),
        ],
        out_specs=[
            pl.BlockSpec((B, D), lambda i: (i, 0)),
            pl.BlockSpec((1, D), lambda i: (0, 0)),
            pl.BlockSpec((1, D), lambda i: (0, 0)),
        ],
        out_shape=[
            jax.ShapeDtypeStruct((NPAD, D), F32),
            jax.ShapeDtypeStruct((1, D), F32),
            jax.ShapeDtypeStruct((1, D), F32),
        ],
    )(p, hp, dp, bias)


def _final(p, hp, dp, bias, B):
    _, NPAD, D = p.shape

    def body(p_ref, hp_ref, dp_ref, b_ref, y_ref):
        pv = p_ref[...]
        dp = dp_ref[...]
        dinv = lax.rsqrt(dp[0][:, 0:1] + dp[1][:, 0:1] + 1.0)
        t = (pv[0] + pv[1] - hp_ref[...]) * dinv + b_ref[...]
        mx = jnp.max(t, axis=1, keepdims=True)
        lse = jnp.log(jnp.sum(jnp.exp(t - mx), axis=1, keepdims=True)) + mx
        y_ref[...] = t - lse

    return pl.pallas_call(
        body,
        grid=(NPAD // B,),
        in_specs=[
            pl.BlockSpec((2, B, D), lambda i: (0, i, 0)),
            pl.BlockSpec((B, D), lambda i: (i, 0)),
            pl.BlockSpec((2, B, 8), lambda i: (0, i, 0)),
            pl.BlockSpec((1, D), lambda i: (0, 0)),
        ],
        out_specs=pl.BlockSpec((B, D), lambda i: (i, 0)),
        out_shape=jax.ShapeDtypeStruct((NPAD, D), F32),
    )(p, hp, dp, bias)


def kernel(input, adj_t, W0, b0, W1, b1, W2, b2, g0, be0, g1, be1):
    N, D = input.shape
    E = adj_t.shape[1]
    NW = NC * NS
    NPAD = -(-N // (NW * 8)) * (NW * 8)
    EPT = E // NW
    assert E % NW == 0 and EPT % 8 == 0
    CH = max(c for c in range(8, 129, 8) if EPT % c == 0)
    B = 2048 if NPAD % 2048 == 0 else (1024 if NPAD % 1024 == 0 else 512)

    NCH = EPT // CH
    adj3 = adj_t.reshape(2, NW, NCH, CH).transpose(1, 2, 0, 3)
    dst3 = adj_t[1].reshape(NW, NCH, CH)
    xpad = jnp.pad(input, ((0, NPAD - N), (0, 0)))
    zcol = jnp.zeros((NPAD, 128), F32)
    ones = jnp.ones((CH, 128), F32)

    deg_fn = _make_deg(E, NPAD, CH)
    agg_fn = _make_agg(E, NPAD, D, CH)

    dp = deg_fn(dst3, ones, zcol)[:, :, :8]

    b0r, b1r, b2r = (b.reshape(1, D) for b in (b0, b1, b2))
    g0r, g1r = g0.reshape(1, D), g1.reshape(1, D)
    be0r, be1r = be0.reshape(1, D), be1.reshape(1, D)

    hp = _matmul0(xpad, W0, dp, B)
    p = agg_fn(hp, adj3)
    t, s, ss = _combine(p, hp, dp, b0r, N, B)
    hp = _matmul_bn(t, s, ss, g0r, be0r, W1, dp, N, B)
    p = agg_fn(hp, adj3)
    t, s, ss = _combine(p, hp, dp, b1r, N, B)
    hp = _matmul_bn(t, s, ss, g1r, be1r, W2, dp, N, B)
    p = agg_fn(hp, adj3)
    y = _final(p, hp, dp, b2r, B)
    return y[:N]
